# baseline (speedup 1.0000x reference)
import jax
import jax.numpy as jnp
from jax import lax
from jax.experimental import pallas as pl
from jax.experimental.pallas import tpu as pltpu
from jax.experimental.pallas import tpu_sc as plsc

N = 10000
E = 320000
D = 128
EPSV = 1e-16

NC = 2
NS = 16
NW = NC * NS
LANES = 16
CH = 128
NCHUNK = E // CH
SPAN = 80
PAIRS = SPAN // 2
EPAD = NW * SPAN * CH
HALF = SPAN // 2
TRASH = 256
STRIPE = 624
TAIL = N - NS * STRIPE

_NEG_LOG_P_MAX = 27.631021


def _mm_body(x_ref, w_ref, o_ref):
    o_ref[...] = jnp.dot(x_ref[...], w_ref[...],
                         preferred_element_type=jnp.float32)


def _matmul(x, w):
    return pl.pallas_call(
        _mm_body,
        grid=(10,),
        in_specs=[
            pl.BlockSpec((N // 10, D), lambda i: (i, 0)),
            pl.BlockSpec((D, D), lambda i: (0, 0)),
        ],
        out_specs=pl.BlockSpec((N // 10, D), lambda i: (i, 0)),
        out_shape=jax.ShapeDtypeStruct((N, D), jnp.float32),
    )(x, w)


def _comb_body(p0_ref, p1_ref, h_ref, o_ref):
    o_ref[...] = p0_ref[...] + p1_ref[...] - h_ref[...]


def _combine(p0, p1, h):
    spec = pl.BlockSpec((N // 10, D), lambda i: (i, 0))
    return pl.pallas_call(
        _comb_body,
        grid=(10,),
        in_specs=[spec, spec, spec],
        out_specs=spec,
        out_shape=jax.ShapeDtypeStruct((N, D), jnp.float32),
    )(p0, p1, h)


def _sc_params():
    return dict(
        mesh=plsc.VectorSubcoreMesh(core_axis_name="c", subcore_axis_name="s"),
        compiler_params=pltpu.CompilerParams(needs_layout_passes=False))


def _sc_scatter_body(h_hbm, src_hbm, dst_hbm, mr_hbm, mc_hbm, part_hbm,
                     accum, sstage, dstage, mstage, gidx, gdst,
                     rows0, mcbuf, sem0):
    cid = lax.axis_index("c")
    sid = lax.axis_index("s")
    w = cid * NS + sid
    span0 = w * (SPAN * CH)

    pltpu.sync_copy(h_hbm.at[pl.ds(sid * STRIPE, STRIPE)],
                    accum.at[pl.ds(sid * STRIPE, STRIPE)])

    @pl.when(sid == 0)
    def _():
        pltpu.sync_copy(h_hbm.at[pl.ds(NS * STRIPE, TAIL)],
                        accum.at[pl.ds(NS * STRIPE, TAIL)])

    pltpu.sync_copy(mc_hbm, mcbuf)
    plsc.subcore_barrier()

    mc16 = mcbuf[...]
    lane = lax.iota(jnp.int32, LANES)

    def fire(rows, sem):
        pltpu.async_copy(h_hbm.at[gidx.at[0]], rows, sem).wait()
        pltpu.sync_copy(rows, accum.at[gdst.at[0]], add=True)

    def rotate():
        for j in range(CH // LANES):
            gidx[0, pl.ds(j * LANES, LANES)] = gidx[1, pl.ds(j * LANES, LANES)]
            gdst[0, pl.ds(j * LANES, LANES)] = gdst[1, pl.ds(j * LANES, LANES)]

    cnt = jnp.int32(0)
    for half in range(2):
        hbase = span0 + half * (HALF * CH)
        pltpu.sync_copy(src_hbm.at[pl.ds(hbase, HALF * CH)], sstage)
        pltpu.sync_copy(dst_hbm.at[pl.ds(hbase, HALF * CH)], dstage)
        pltpu.sync_copy(mr_hbm.at[pl.ds(hbase, HALF * CH)], mstage)

        def chunk_body(c, cnt):
            for g in range(CH // LANES):
                mr16 = mstage[pl.ds(c * CH + g * LANES, LANES)]
                s16 = sstage[pl.ds(c * CH + g * LANES, LANES)]
                d16 = dstage[pl.ds(c * CH + g * LANES, LANES)]
                keep = mr16 >= mc16
                k01 = jnp.where(keep, 1, 0).astype(jnp.int32)
                pos = cnt + (plsc.cumsum(k01) - k01)
                plsc.store_scatter(gidx, [pos >> 7, pos & (CH - 1)], s16,
                                   mask=keep)
                plsc.store_scatter(gdst, [pos >> 7, pos & (CH - 1)], d16,
                                   mask=keep)
                cnt = cnt + jnp.sum(k01)

            @pl.when(cnt >= CH)
            def _():
                fire(rows0, sem0)
                rotate()

            return jnp.where(cnt >= CH, cnt - CH, cnt)

        cnt = lax.fori_loop(0, HALF, chunk_body, cnt)

    cntv = jnp.full((LANES,), cnt, jnp.int32)
    for j in range(CH // LANES):
        posj = j * LANES + lane
        tail = posj >= cntv
        cur_i = gidx[0, pl.ds(j * LANES, LANES)]
        cur_d = gdst[0, pl.ds(j * LANES, LANES)]
        gidx[0, pl.ds(j * LANES, LANES)] = jnp.where(tail, posj, cur_i)
        gdst[0, pl.ds(j * LANES, LANES)] = jnp.where(
            tail, N + ((w * 8 + posj) % TRASH), cur_d)
    fire(rows0, sem0)

    plsc.subcore_barrier()
    pltpu.sync_copy(accum.at[pl.ds(sid * STRIPE, STRIPE)],
                    part_hbm.at[cid, pl.ds(sid * STRIPE, STRIPE)])

    @pl.when(sid == 0)
    def _():
        pltpu.sync_copy(accum.at[pl.ds(NS * STRIPE, TAIL)],
                        part_hbm.at[cid, pl.ds(NS * STRIPE, TAIL)])


def _sc_scatter(h, src, dst, mr, mc16):
    f = pl.kernel(
        _sc_scatter_body,
        out_type=jax.ShapeDtypeStruct((NC, N, D), jnp.float32),
        scratch_types=[
            pltpu.VMEM_SHARED((N + TRASH, D), jnp.float32),
            pltpu.VMEM((HALF * CH,), jnp.int32),
            pltpu.VMEM((HALF * CH,), jnp.int32),
            pltpu.VMEM((HALF * CH,), jnp.float32),
            pltpu.VMEM((2, CH), jnp.int32),
            pltpu.VMEM((2, CH), jnp.int32),
            pltpu.VMEM((CH, D), jnp.float32),
            pltpu.VMEM((LANES,), jnp.float32),
            pltpu.SemaphoreType.DMA,
        ],
        **_sc_params(),
    )
    return f(h, src, dst, mr, mc16)


def _softplus_neg(l16):
    u = jnp.exp(-jnp.abs(l16))
    s = u / (2.0 + u)
    s2 = s * s
    log1p_u = s * (2.0 + s2 * (2.0 / 3.0 + s2 * (2.0 / 5.0 + s2 * (2.0 / 7.0))))
    val = jnp.maximum(-l16, 0.0) + log1p_u
    return jnp.minimum(val, _NEG_LOG_P_MAX)


def _sc_loss_body(zt_hbm, src_hbm, dst_hbm, mr_hbm, mc_hbm, out_hbm,
                  sstage, dstage, mstage, rows_a0, rows_b0, rows_a1, rows_b1,
                  mcbuf, accbuf, sem_a0, sem_b0, sem_a1, sem_b1):
    cid = lax.axis_index("c")
    sid = lax.axis_index("s")
    w = cid * NS + sid
    span0 = w * (SPAN * CH)

    pltpu.sync_copy(src_hbm.at[pl.ds(span0, SPAN * CH)], sstage)
    pltpu.sync_copy(dst_hbm.at[pl.ds(span0, SPAN * CH)], dstage)
    pltpu.sync_copy(mr_hbm.at[pl.ds(span0, SPAN * CH)], mstage)
    pltpu.sync_copy(mc_hbm, mcbuf)
    mc16 = mcbuf[...]
    lane = lax.iota(jnp.int32, LANES)
    zero16 = jnp.zeros((LANES,), jnp.float32)

    def gather(c, rows_a, rows_b, sem_a, sem_b):
        pltpu.async_copy(
            zt_hbm.at[sstage.at[pl.ds(c * CH, CH)]], rows_a, sem_a)
        pltpu.async_copy(
            zt_hbm.at[dstage.at[pl.ds(c * CH, CH)]], rows_b, sem_b)

    def wait(rows_a, rows_b, sem_a, sem_b):
        pltpu.make_async_copy(
            zt_hbm.at[sstage.at[pl.ds(0, CH)]], rows_a, sem_a).wait()
        pltpu.make_async_copy(
            zt_hbm.at[sstage.at[pl.ds(0, CH)]], rows_b, sem_b).wait()

    def compute(c, rows_a, rows_b, acc):
        gc = w * SPAN + c
        validf = jnp.where(gc < NCHUNK, 1.0, 0.0).astype(jnp.float32)

        NG = 4

        def quad_groups(gp, acc):
            rowv = [NG * gp * LANES + q * LANES + lane for q in range(NG)]
            d = [zero16] * (2 * NG)
            for k in range(0, D, 2):
                for i in range(2):
                    if k + i + LANES - 1 < D:
                        kv = lane + (k + i)
                    else:
                        kv = (lane + (k + i)) & (D - 1)
                    for q in range(NG):
                        d[2 * q + i] = d[2 * q + i] + (
                            plsc.load_gather(rows_a, [rowv[q], kv])
                            * plsc.load_gather(rows_b, [rowv[q], kv]))
            contrib = zero16
            for q in range(NG):
                dotq = d[2 * q] + d[2 * q + 1]
                mrq = mstage[pl.ds(c * CH + (NG * gp + q) * LANES, LANES)]
                mq = jnp.where(mrq < mc16, validf, 0.0)
                contrib = contrib + mq * _softplus_neg(dotq)
            return acc + contrib

        return lax.fori_loop(0, CH // (NG * LANES), quad_groups, acc)

    gather(0, rows_a0, rows_b0, sem_a0, sem_b0)
    gather(1, rows_a1, rows_b1, sem_a1, sem_b1)

    def pair_body(p, acc):
        c0 = 2 * p
        wait(rows_a0, rows_b0, sem_a0, sem_b0)
        acc = compute(c0, rows_a0, rows_b0, acc)

        @pl.when(p < PAIRS - 1)
        def _():
            gather(c0 + 2, rows_a0, rows_b0, sem_a0, sem_b0)

        c1 = 2 * p + 1
        wait(rows_a1, rows_b1, sem_a1, sem_b1)
        acc = compute(c1, rows_a1, rows_b1, acc)

        @pl.when(p < PAIRS - 1)
        def _():
            gather(c1 + 2, rows_a1, rows_b1, sem_a1, sem_b1)

        return acc

    acc = lax.fori_loop(0, PAIRS, pair_body, zero16)
    accbuf[...] = acc
    pltpu.sync_copy(accbuf, out_hbm.at[w])


def _sc_loss(zt, src, dst, mr, mc16):
    f = pl.kernel(
        _sc_loss_body,
        out_type=jax.ShapeDtypeStruct((NW, LANES), jnp.float32),
        scratch_types=[
            pltpu.VMEM((SPAN * CH,), jnp.int32),
            pltpu.VMEM((SPAN * CH,), jnp.int32),
            pltpu.VMEM((SPAN * CH,), jnp.float32),
            pltpu.VMEM((CH, D), jnp.float32),
            pltpu.VMEM((CH, D), jnp.float32),
            pltpu.VMEM((CH, D), jnp.float32),
            pltpu.VMEM((CH, D), jnp.float32),
            pltpu.VMEM((LANES,), jnp.float32),
            pltpu.VMEM((LANES,), jnp.float32),
            pltpu.SemaphoreType.DMA,
            pltpu.SemaphoreType.DMA,
            pltpu.SemaphoreType.DMA,
            pltpu.SemaphoreType.DMA,
        ],
        **_sc_params(),
    )
    return f(zt, src, dst, mr, mc16)


def kernel(x, edge_index, t_rand, mask_rand, W):
    t = (1.0 - EPSV) * t_rand[0] + EPSV
    sigma = -jnp.log1p(-(1.0 - EPSV) * t)
    dsigma = (1.0 - EPSV) / (1.0 - (1.0 - EPSV) * t)
    move_chance = 1.0 - jnp.exp(-sigma)
    coef = dsigma / jnp.expm1(sigma)
    mc16 = jnp.full((LANES,), move_chance, jnp.float32)

    pad = EPAD - E
    pad_idx = (jnp.arange(pad, dtype=jnp.int32) * 53) % N
    src = jnp.concatenate([edge_index[0].astype(jnp.int32), pad_idx])
    dst = jnp.concatenate([edge_index[1].astype(jnp.int32), pad_idx])
    mr = jnp.concatenate(
        [mask_rand.astype(jnp.float32), jnp.full((pad,), -1.0, jnp.float32)])

    h = _matmul(x, W)
    part = _sc_scatter(h, src, dst, mr, mc16)
    zt = _combine(part[0], part[1], h)
    partials = _sc_loss(zt, src, dst, mr, mc16)
    return coef * jnp.sum(partials)


# speedup vs baseline: 1.9271x; 1.9271x over previous
import jax
import jax.numpy as jnp
from jax import lax
from jax.experimental import pallas as pl
from jax.experimental.pallas import tpu as pltpu
from jax.experimental.pallas import tpu_sc as plsc

N = 10000
E = 320000
D = 128
EPSV = 1e-16

NC = 2
NS = 16
NW = NC * NS
LANES = 16
CH = 128
NCHUNK = E // CH
SPAN = 80
PAIRS = SPAN // 2
EPAD = NW * SPAN * CH
HALF = SPAN // 2
TRASH = 256
STRIPE = 624
TAIL = N - NS * STRIPE

_NEG_LOG_P_MAX = 27.631021


def _mm_body(x_ref, w_ref, o_ref):
    o_ref[...] = jnp.dot(x_ref[...], w_ref[...],
                         preferred_element_type=jnp.float32)


def _matmul(x, w):
    return pl.pallas_call(
        _mm_body,
        grid=(10,),
        in_specs=[
            pl.BlockSpec((N // 10, D), lambda i: (i, 0)),
            pl.BlockSpec((D, D), lambda i: (0, 0)),
        ],
        out_specs=pl.BlockSpec((N // 10, D), lambda i: (i, 0)),
        out_shape=jax.ShapeDtypeStruct((N, D), jnp.float32),
    )(x, w)


def _comb_body(p0_ref, p1_ref, h_ref, o_ref):
    o_ref[...] = p0_ref[...] + p1_ref[...] - h_ref[...]


def _combine(p0, p1, h):
    spec = pl.BlockSpec((N // 10, D), lambda i: (i, 0))
    return pl.pallas_call(
        _comb_body,
        grid=(10,),
        in_specs=[spec, spec, spec],
        out_specs=spec,
        out_shape=jax.ShapeDtypeStruct((N, D), jnp.float32),
    )(p0, p1, h)


def _sc_params():
    return dict(
        mesh=plsc.VectorSubcoreMesh(core_axis_name="c", subcore_axis_name="s"),
        compiler_params=pltpu.CompilerParams(needs_layout_passes=False))


def _sc_scatter_body(h_hbm, src_hbm, dst_hbm, mr_hbm, mc_hbm, part_hbm,
                     accum, sstage, dstage, mstage, gidx, gdst,
                     rows0, mcbuf, sem0):
    cid = lax.axis_index("c")
    sid = lax.axis_index("s")
    w = cid * NS + sid
    span0 = w * (SPAN * CH)

    pltpu.sync_copy(h_hbm.at[pl.ds(sid * STRIPE, STRIPE)],
                    accum.at[pl.ds(sid * STRIPE, STRIPE)])

    @pl.when(sid == 0)
    def _():
        pltpu.sync_copy(h_hbm.at[pl.ds(NS * STRIPE, TAIL)],
                        accum.at[pl.ds(NS * STRIPE, TAIL)])

    pltpu.sync_copy(mc_hbm, mcbuf)
    plsc.subcore_barrier()

    mc16 = mcbuf[...]
    lane = lax.iota(jnp.int32, LANES)

    def fire(rows, sem):
        pltpu.async_copy(h_hbm.at[gidx.at[0]], rows, sem).wait()
        pltpu.sync_copy(rows, accum.at[gdst.at[0]], add=True)

    def rotate():
        for j in range(CH // LANES):
            gidx[0, pl.ds(j * LANES, LANES)] = gidx[1, pl.ds(j * LANES, LANES)]
            gdst[0, pl.ds(j * LANES, LANES)] = gdst[1, pl.ds(j * LANES, LANES)]

    cnt = jnp.int32(0)
    for half in range(2):
        hbase = span0 + half * (HALF * CH)
        pltpu.sync_copy(src_hbm.at[pl.ds(hbase, HALF * CH)], sstage)
        pltpu.sync_copy(dst_hbm.at[pl.ds(hbase, HALF * CH)], dstage)
        pltpu.sync_copy(mr_hbm.at[pl.ds(hbase, HALF * CH)], mstage)

        def chunk_body(c, cnt):
            for g in range(CH // LANES):
                mr16 = mstage[pl.ds(c * CH + g * LANES, LANES)]
                s16 = sstage[pl.ds(c * CH + g * LANES, LANES)]
                d16 = dstage[pl.ds(c * CH + g * LANES, LANES)]
                keep = mr16 >= mc16
                k01 = jnp.where(keep, 1, 0).astype(jnp.int32)
                pos = cnt + (plsc.cumsum(k01) - k01)
                plsc.store_scatter(gidx, [pos >> 7, pos & (CH - 1)], s16,
                                   mask=keep)
                plsc.store_scatter(gdst, [pos >> 7, pos & (CH - 1)], d16,
                                   mask=keep)
                cnt = cnt + jnp.sum(k01)

            @pl.when(cnt >= CH)
            def _():
                fire(rows0, sem0)
                rotate()

            return jnp.where(cnt >= CH, cnt - CH, cnt)

        cnt = lax.fori_loop(0, HALF, chunk_body, cnt)

    cntv = jnp.full((LANES,), cnt, jnp.int32)
    for j in range(CH // LANES):
        posj = j * LANES + lane
        tail = posj >= cntv
        cur_i = gidx[0, pl.ds(j * LANES, LANES)]
        cur_d = gdst[0, pl.ds(j * LANES, LANES)]
        gidx[0, pl.ds(j * LANES, LANES)] = jnp.where(tail, posj, cur_i)
        gdst[0, pl.ds(j * LANES, LANES)] = jnp.where(
            tail, N + ((w * 8 + posj) % TRASH), cur_d)
    fire(rows0, sem0)

    plsc.subcore_barrier()
    pltpu.sync_copy(accum.at[pl.ds(sid * STRIPE, STRIPE)],
                    part_hbm.at[cid, pl.ds(sid * STRIPE, STRIPE)])

    @pl.when(sid == 0)
    def _():
        pltpu.sync_copy(accum.at[pl.ds(NS * STRIPE, TAIL)],
                        part_hbm.at[cid, pl.ds(NS * STRIPE, TAIL)])


def _sc_scatter(h, src, dst, mr, mc16):
    f = pl.kernel(
        _sc_scatter_body,
        out_type=jax.ShapeDtypeStruct((NC, N, D), jnp.float32),
        scratch_types=[
            pltpu.VMEM_SHARED((N + TRASH, D), jnp.float32),
            pltpu.VMEM((HALF * CH,), jnp.int32),
            pltpu.VMEM((HALF * CH,), jnp.int32),
            pltpu.VMEM((HALF * CH,), jnp.float32),
            pltpu.VMEM((2, CH), jnp.int32),
            pltpu.VMEM((2, CH), jnp.int32),
            pltpu.VMEM((CH, D), jnp.float32),
            pltpu.VMEM((LANES,), jnp.float32),
            pltpu.SemaphoreType.DMA,
        ],
        **_sc_params(),
    )
    return f(h, src, dst, mr, mc16)


def _softplus_neg(l16):
    u = jnp.exp(-jnp.abs(l16))
    s = u / (2.0 + u)
    s2 = s * s
    log1p_u = s * (2.0 + s2 * (2.0 / 3.0 + s2 * (2.0 / 5.0 + s2 * (2.0 / 7.0))))
    val = jnp.maximum(-l16, 0.0) + log1p_u
    return jnp.minimum(val, _NEG_LOG_P_MAX)


def _sc_loss_body(zt_hbm, src_hbm, dst_hbm, mr_hbm, mc_hbm, out_hbm,
                  sstage, dstage, mstage, rows_a0, rows_b0, rows_a1, rows_b1,
                  mcbuf, accbuf, sem_a0, sem_b0, sem_a1, sem_b1):
    cid = lax.axis_index("c")
    sid = lax.axis_index("s")
    w = cid * NS + sid
    span0 = w * (SPAN * CH)

    pltpu.sync_copy(src_hbm.at[pl.ds(span0, SPAN * CH)], sstage)
    pltpu.sync_copy(dst_hbm.at[pl.ds(span0, SPAN * CH)], dstage)
    pltpu.sync_copy(mr_hbm.at[pl.ds(span0, SPAN * CH)], mstage)
    pltpu.sync_copy(mc_hbm, mcbuf)
    mc16 = mcbuf[...]
    lane = lax.iota(jnp.int32, LANES)
    zero16 = jnp.zeros((LANES,), jnp.float32)

    def gather(c, rows_a, rows_b, sem_a, sem_b):
        pltpu.async_copy(
            zt_hbm.at[sstage.at[pl.ds(c * CH, CH)]], rows_a, sem_a)
        pltpu.async_copy(
            zt_hbm.at[dstage.at[pl.ds(c * CH, CH)]], rows_b, sem_b)

    def wait(rows_a, rows_b, sem_a, sem_b):
        pltpu.make_async_copy(
            zt_hbm.at[sstage.at[pl.ds(0, CH)]], rows_a, sem_a).wait()
        pltpu.make_async_copy(
            zt_hbm.at[sstage.at[pl.ds(0, CH)]], rows_b, sem_b).wait()

    def compute(c, rows_a, rows_b, acc):
        gc = w * SPAN + c
        validf = jnp.where(gc < NCHUNK, 1.0, 0.0).astype(jnp.float32)

        NG = 2

        def quad_groups(gp, acc):
            rowv = [NG * gp * LANES + q * LANES + lane for q in range(NG)]
            d = [zero16] * (4 * NG)
            for k in range(0, D, 4):
                for i in range(4):
                    if k + i + LANES - 1 < D:
                        kv = lane + (k + i)
                    else:
                        kv = (lane + (k + i)) & (D - 1)
                    for q in range(NG):
                        d[4 * q + i] = d[4 * q + i] + (
                            plsc.load_gather(rows_a, [rowv[q], kv])
                            * plsc.load_gather(rows_b, [rowv[q], kv]))
            contrib = zero16
            for q in range(NG):
                dotq = (d[4 * q] + d[4 * q + 1]) + (d[4 * q + 2] + d[4 * q + 3])
                mrq = mstage[pl.ds(c * CH + (NG * gp + q) * LANES, LANES)]
                mq = jnp.where(mrq < mc16, validf, 0.0)
                contrib = contrib + mq * _softplus_neg(dotq)
            return acc + contrib

        return lax.fori_loop(0, CH // (NG * LANES), quad_groups, acc)

    gather(0, rows_a0, rows_b0, sem_a0, sem_b0)
    gather(1, rows_a1, rows_b1, sem_a1, sem_b1)

    def pair_body(p, acc):
        c0 = 2 * p
        wait(rows_a0, rows_b0, sem_a0, sem_b0)
        acc = compute(c0, rows_a0, rows_b0, acc)

        @pl.when(p < PAIRS - 1)
        def _():
            gather(c0 + 2, rows_a0, rows_b0, sem_a0, sem_b0)

        c1 = 2 * p + 1
        wait(rows_a1, rows_b1, sem_a1, sem_b1)
        acc = compute(c1, rows_a1, rows_b1, acc)

        @pl.when(p < PAIRS - 1)
        def _():
            gather(c1 + 2, rows_a1, rows_b1, sem_a1, sem_b1)

        return acc

    acc = lax.fori_loop(0, PAIRS, pair_body, zero16)
    accbuf[...] = acc
    pltpu.sync_copy(accbuf, out_hbm.at[w])


def _sc_loss(zt, src, dst, mr, mc16):
    f = pl.kernel(
        _sc_loss_body,
        out_type=jax.ShapeDtypeStruct((NW, LANES), jnp.float32),
        scratch_types=[
            pltpu.VMEM((SPAN * CH,), jnp.int32),
            pltpu.VMEM((SPAN * CH,), jnp.int32),
            pltpu.VMEM((SPAN * CH,), jnp.float32),
            pltpu.VMEM((CH, D), jnp.float32),
            pltpu.VMEM((CH, D), jnp.float32),
            pltpu.VMEM((CH, D), jnp.float32),
            pltpu.VMEM((CH, D), jnp.float32),
            pltpu.VMEM((LANES,), jnp.float32),
            pltpu.VMEM((LANES,), jnp.float32),
            pltpu.SemaphoreType.DMA,
            pltpu.SemaphoreType.DMA,
            pltpu.SemaphoreType.DMA,
            pltpu.SemaphoreType.DMA,
        ],
        **_sc_params(),
    )
    return f(zt, src, dst, mr, mc16)


def kernel(x, edge_index, t_rand, mask_rand, W):
    t = (1.0 - EPSV) * t_rand[0] + EPSV
    sigma = -jnp.log1p(-(1.0 - EPSV) * t)
    dsigma = (1.0 - EPSV) / (1.0 - (1.0 - EPSV) * t)
    move_chance = 1.0 - jnp.exp(-sigma)
    coef = dsigma / jnp.expm1(sigma)
    mc16 = jnp.full((LANES,), move_chance, jnp.float32)

    pad = EPAD - E
    pad_idx = (jnp.arange(pad, dtype=jnp.int32) * 53) % N
    src = jnp.concatenate([edge_index[0].astype(jnp.int32), pad_idx])
    dst = jnp.concatenate([edge_index[1].astype(jnp.int32), pad_idx])
    mr = jnp.concatenate(
        [mask_rand.astype(jnp.float32), jnp.full((pad,), -1.0, jnp.float32)])

    h = _matmul(x, W)
    part = _sc_scatter(h, src, dst, mr, mc16)
    zt = _combine(part[0], part[1], h)
    partials = _sc_loss(zt, src, dst, mr, mc16)
    return coef * jnp.sum(partials)
